# baseline (speedup 1.0000x reference)
import functools

import jax
import jax.numpy as jnp
from jax import lax
from jax.experimental import pallas as pl
from jax.experimental.pallas import tpu as pltpu
from jax.experimental.pallas import tpu_sc as plsc

NUM_EMB = 1_000_000
DIM = 64
BATCH = 4096
SEQ = 200
NTOK = BATCH * SEQ

NC = 2
NS = 16
NW = NC * NS
PER_W = NTOK // NW
CHUNK = 800
NCHUNK = PER_W // CHUNK


def _emb_body(ids_hbm, table_hbm, out_hbm, idx_v, rows_v, sem):
    wid = lax.axis_index("s") * NC + lax.axis_index("c")
    base = wid * PER_W

    def step(i, carry):
        off = base + i * CHUNK
        pltpu.sync_copy(ids_hbm.at[pl.ds(off, CHUNK)], idx_v)
        pltpu.async_copy(table_hbm.at[idx_v], rows_v, sem).wait()
        pltpu.sync_copy(rows_v, out_hbm.at[pl.ds(off, CHUNK)])
        return carry

    lax.fori_loop(0, NCHUNK, step, 0)


@jax.jit
def _emb_call(ids_flat, table):
    grid_kernel = pl.kernel(
        _emb_body,
        out_type=jax.ShapeDtypeStruct((NTOK, DIM), jnp.float32),
        mesh=plsc.VectorSubcoreMesh(core_axis_name="c", subcore_axis_name="s"),
        scratch_types=[
            pltpu.VMEM((CHUNK,), jnp.int32),
            pltpu.VMEM((CHUNK, DIM), jnp.float32),
            pltpu.SemaphoreType.DMA,
        ],
        compiler_params=pltpu.CompilerParams(use_tc_tiling_on_sc=False),
    )
    return grid_kernel(ids_flat, table)


def kernel(token_ids, embedding):
    ids_flat = token_ids.reshape(NTOK).astype(jnp.int32)
    out = _emb_call(ids_flat, embedding)
    return out.reshape(BATCH, SEQ, DIM)


# speedup vs baseline: 1.0276x; 1.0276x over previous
import functools

import jax
import jax.numpy as jnp
from jax import lax
from jax.experimental import pallas as pl
from jax.experimental.pallas import tpu as pltpu
from jax.experimental.pallas import tpu_sc as plsc

NUM_EMB = 1_000_000
DIM = 64
BATCH = 4096
SEQ = 200
NTOK = BATCH * SEQ

NC = 2
NS = 16
NW = NC * NS
PER_W = NTOK // NW
CHUNK = 800
NBUF = 2
NCHUNK = PER_W // CHUNK


def _emb_body(ids_hbm, table_hbm, out_hbm, idx_v, rows_v, gsem, ssem):
    wid = lax.axis_index("s") * NC + lax.axis_index("c")
    base = wid * PER_W

    pltpu.sync_copy(ids_hbm.at[pl.ds(base, PER_W)], idx_v)

    def gather(i, b):
        pltpu.async_copy(
            table_hbm.at[idx_v.at[pl.ds(i * CHUNK, CHUNK)]],
            rows_v.at[b], gsem.at[b])

    def store(i, b):
        pltpu.async_copy(
            rows_v.at[b], out_hbm.at[pl.ds(base + i * CHUNK, CHUNK)],
            ssem.at[b])

    def wait_gather(i, b):
        pltpu.make_async_copy(
            table_hbm.at[idx_v.at[pl.ds(i * CHUNK, CHUNK)]],
            rows_v.at[b], gsem.at[b]).wait()

    def wait_store(i, b):
        pltpu.make_async_copy(
            rows_v.at[b], out_hbm.at[pl.ds(base + i * CHUNK, CHUNK)],
            ssem.at[b]).wait()

    gather(0, 0)

    def step(i, carry):
        b = lax.rem(i, NBUF)
        nb = lax.rem(i + 1, NBUF)

        @pl.when(i + 1 < NCHUNK)
        def _():
            @pl.when(i + 1 >= NBUF)
            def _():
                wait_store(i + 1 - NBUF, nb)
            gather(i + 1, nb)

        wait_gather(i, b)
        store(i, b)
        return carry

    lax.fori_loop(0, NCHUNK, step, 0)
    wait_store(NCHUNK - NBUF, lax.rem(NCHUNK - NBUF, NBUF))
    wait_store(NCHUNK - 1, lax.rem(NCHUNK - 1, NBUF))


@jax.jit
def _emb_call(ids_flat, table):
    grid_kernel = pl.kernel(
        _emb_body,
        out_type=jax.ShapeDtypeStruct((NTOK, DIM), jnp.float32),
        mesh=plsc.VectorSubcoreMesh(core_axis_name="c", subcore_axis_name="s"),
        scratch_types=[
            pltpu.VMEM((PER_W,), jnp.int32),
            pltpu.VMEM((NBUF, CHUNK, DIM), jnp.float32),
            pltpu.SemaphoreType.DMA((NBUF,)),
            pltpu.SemaphoreType.DMA((NBUF,)),
        ],
        compiler_params=pltpu.CompilerParams(use_tc_tiling_on_sc=False),
    )
    return grid_kernel(ids_flat, table)


def kernel(token_ids, embedding):
    ids_flat = token_ids.reshape(NTOK).astype(jnp.int32)
    out = _emb_call(ids_flat, embedding)
    return out.reshape(BATCH, SEQ, DIM)
